# baseline (speedup 1.0000x reference)
import functools

import jax
import jax.numpy as jnp
from jax import lax
from jax.experimental import pallas as pl
from jax.experimental.pallas import tpu as pltpu
from jax.experimental.pallas import tpu_sc as plsc

NUM_CORES = 2
NUM_SUBCORES = 16
LANES = 16
NUM_WORKERS = NUM_CORES * NUM_SUBCORES

ROWS = 16384
COLS = 200
ROWS_PER_WORKER = ROWS // NUM_WORKERS
BLOCKS_PER_WORKER = ROWS_PER_WORKER // LANES


@functools.partial(
    pl.kernel,
    out_type=jax.ShapeDtypeStruct((ROWS,), jnp.float32),
    mesh=plsc.VectorSubcoreMesh(core_axis_name="c", subcore_axis_name="s"),
    compiler_params=pltpu.CompilerParams(needs_layout_passes=False),
    scratch_types=[
        pltpu.VMEM((ROWS_PER_WORKER * COLS,), jnp.int32),
        pltpu.VMEM((ROWS_PER_WORKER,), jnp.float32),
        pltpu.VMEM((ROWS_PER_WORKER,), jnp.float32),
        pltpu.VMEM((LANES,), jnp.float32),
    ],
)
def _sae_add(species_hbm, energies_hbm, table_hbm, out_hbm,
             sp_v, en_v, out_v, tab_v):
    wid = lax.axis_index("s") * NUM_CORES + lax.axis_index("c")
    base = wid * ROWS_PER_WORKER

    pltpu.sync_copy(table_hbm, tab_v)
    pltpu.sync_copy(energies_hbm.at[pl.ds(base, ROWS_PER_WORKER)], en_v)
    pltpu.sync_copy(
        species_hbm.at[pl.ds(base * COLS, ROWS_PER_WORKER * COLS)], sp_v)

    lane = jnp.arange(LANES, dtype=jnp.int32)
    for b in range(BLOCKS_PER_WORKER):
        rowoff = (lane + b * LANES) * COLS

        def body(j, carry):
            acc, colv = carry
            sv = plsc.load_gather(sp_v, [rowoff + colv])
            tv = plsc.load_gather(tab_v, [sv & 15])
            return acc + tv, colv + 1

        acc, _ = lax.fori_loop(
            0, COLS, body,
            (jnp.zeros((LANES,), jnp.float32), jnp.zeros((LANES,), jnp.int32)),
            unroll=8,
        )
        out_v[pl.ds(b * LANES, LANES)] = acc + en_v[pl.ds(b * LANES, LANES)]

    pltpu.sync_copy(out_v, out_hbm.at[pl.ds(base, ROWS_PER_WORKER)])


def kernel(species, energies, self_energies):
    table16 = jnp.pad(self_energies.astype(jnp.float32), (0, 16 - 10))
    species_flat = species.astype(jnp.int32).reshape(-1)
    new_energies = _sae_add(species_flat, energies, table16)
    return (species, new_energies)
